# baseline (speedup 1.0000x reference)
import dataclasses
import functools

import jax
import jax.numpy as jnp
from jax import lax
from jax.experimental import pallas as pl
from jax.experimental.pallas import tpu as pltpu
from jax.experimental.pallas import tpu_sc as plsc

_NC = 2
_NS = 16
_NW = _NC * _NS
_CW = 128
_LANES = 16


def _mesh():
    return plsc.VectorSubcoreMesh(core_axis_name="c", subcore_axis_name="s")


def _vperm(x, idx):
    dn = lax.GatherDimensionNumbers(
        offset_dims=(), collapsed_slice_dims=(0,), start_index_map=(0,))
    return lax.gather(x, idx[:, None], dn, (1,),
                      mode=lax.GatherScatterMode.PROMISE_IN_BOUNDS)


def _sc_params():
    cp = pltpu.CompilerParams()
    if "needs_layout_passes" in pltpu.CompilerParams.__dataclass_fields__:
        cp = dataclasses.replace(cp, needs_layout_passes=False)
    return cp


def _front_body(num_x_ref, cat_x_ref, emb_W_ref, emb_b_ref, gcn_W_ref,
                vo_ref, fcW0_ref, fc_b_ref,
                xw_num_ref, xw_cat_ref, vout_ref):
    cat_emb = (jnp.sum(cat_x_ref[...][:, :, None] * emb_W_ref[...], axis=1)
               + emb_b_ref[...])
    gcn_W = gcn_W_ref[...]
    xw_num_ref[...] = jnp.dot(num_x_ref[...], gcn_W,
                              preferred_element_type=jnp.float32)
    xw_cat_ref[...] = jnp.dot(cat_emb, gcn_W,
                              preferred_element_type=jnp.float32)
    vout_ref[...] = (jnp.dot(vo_ref[...], fcW0_ref[...],
                             preferred_element_type=jnp.float32)
                     + fc_b_ref[...][None, :])


def _front(num_x, cat_x, emb_W, emb_b, gcn_W, vanilla_out, fcW0, fc_b):
    n_cont, _ = num_x.shape
    n_cat = cat_x.shape[0]
    h = gcn_W.shape[1]
    b, ncls = vanilla_out.shape[0], fcW0.shape[1]
    return pl.pallas_call(
        _front_body,
        out_shape=(
            jax.ShapeDtypeStruct((n_cont, h), jnp.float32),
            jax.ShapeDtypeStruct((n_cat, h), jnp.float32),
            jax.ShapeDtypeStruct((b, ncls), jnp.float32),
        ),
    )(num_x, cat_x, emb_W, emb_b, gcn_W, vanilla_out, fcW0, fc_b)


def _deg_body(col_hbm, w_hbm, zeros_hbm, out_hbm, col_v, w_v, deg_sh, sem):
    cid = lax.axis_index("c")
    sid = lax.axis_index("s")
    wid = sid * _NC + cid
    epw = col_v.shape[0]
    nch = epw // _CW

    @pl.when(sid == 0)
    def _():
        pltpu.sync_copy(zeros_hbm, deg_sh)

    pltpu.sync_copy(col_hbm.at[pl.ds(wid * epw, epw)], col_v)
    pltpu.sync_copy(w_hbm.at[pl.ds(wid * epw, epw)], w_v)
    plsc.subcore_barrier()

    @pl.loop(0, nch)
    def _(k):
        sl = pl.ds(k * _CW, _CW)
        pltpu.async_copy(w_v.at[sl], deg_sh.at[col_v.at[sl]], sem, add=True)

    @pl.loop(0, nch)
    def _(k):
        pltpu.make_async_copy(zeros_hbm.at[pl.ds(0, _CW)],
                              w_v.at[pl.ds(0, _CW)], sem).wait()

    plsc.subcore_barrier()

    @pl.when(sid == 0)
    def _():
        pltpu.sync_copy(deg_sh, out_hbm.at[cid])


def _deg(col, w, zeros_n):
    n = zeros_n.shape[0]
    epw = col.shape[0] // _NW
    kern = pl.kernel(
        _deg_body,
        out_type=jax.ShapeDtypeStruct((_NC, n), jnp.float32),
        mesh=_mesh(),
        scratch_types=[
            pltpu.VMEM((epw,), jnp.int32),
            pltpu.VMEM((epw,), jnp.float32),
            pltpu.VMEM_SHARED((n,), jnp.float32),
            pltpu.SemaphoreType.DMA,
        ],
    )
    return kern(col, w, zeros_n)


def _mid_body(degp_ref, xw_ref, dis_ref, y_ref):
    deg = 1.0 + degp_ref[0, :] + degp_ref[1, :]
    dis = lax.rsqrt(deg)
    dis_ref[...] = dis
    y_ref[...] = dis[:, None] * xw_ref[...]


def _mid(degp, xw):
    n, h = xw.shape
    return pl.pallas_call(
        _mid_body,
        out_shape=(
            jax.ShapeDtypeStruct((n,), jnp.float32),
            jax.ShapeDtypeStruct((n, h), jnp.float32),
        ),
    )(degp, xw)


def _msg(y_flat, row, col, w, zeros_n):
    n = zeros_n.shape[0]
    nch = row.shape[0] // (_NW * _CW)
    epw = nch * _CW

    def body(y_hbm, row_hbm, col_hbm, w_hbm, zeros_hbm, out_hbm,
             y_v, row_v, col_v, w_v, u0, u1, u2, u3, s0, s1, s2, s3, sem):
        cid = lax.axis_index("c")
        sid = lax.axis_index("s")
        wid = sid * _NC + cid

        stage = [
            pltpu.async_copy(y_hbm, y_v, sem),
            pltpu.async_copy(row_hbm.at[pl.ds(wid * epw, epw)], row_v, sem),
            pltpu.async_copy(col_hbm.at[pl.ds(wid * epw, epw)], col_v, sem),
            pltpu.async_copy(w_hbm.at[pl.ds(wid * epw, epw)], w_v, sem),
        ]

        @pl.when(sid == 0)
        def _():
            pltpu.sync_copy(zeros_hbm, s0)
            pltpu.sync_copy(zeros_hbm, s1)
            pltpu.sync_copy(zeros_hbm, s2)
            pltpu.sync_copy(zeros_hbm, s3)

        for d in stage:
            d.wait()
        plsc.subcore_barrier()

        @pl.loop(0, nch)
        def _(k):
            sl = pl.ds(k * _CW, _CW)

            @pl.loop(0, _CW, step=_LANES)
            def _(t):
                e = k * _CW + t
                r16 = row_v[pl.ds(e, _LANES)]
                i16 = r16 << 2
                w16 = w_v[pl.ds(e, _LANES)]
                u0[pl.ds(e, _LANES)] = plsc.load_gather(y_v, [i16]) * w16
                u1[pl.ds(e, _LANES)] = plsc.load_gather(y_v, [i16 + 1]) * w16
                u2[pl.ds(e, _LANES)] = plsc.load_gather(y_v, [i16 + 2]) * w16
                u3[pl.ds(e, _LANES)] = plsc.load_gather(y_v, [i16 + 3]) * w16

            idx = col_v.at[sl]
            pltpu.async_copy(u0.at[sl], s0.at[idx], sem, add=True)
            pltpu.async_copy(u1.at[sl], s1.at[idx], sem, add=True)
            pltpu.async_copy(u2.at[sl], s2.at[idx], sem, add=True)
            pltpu.async_copy(u3.at[sl], s3.at[idx], sem, add=True)

        @pl.loop(0, 4 * nch)
        def _(k):
            pltpu.make_async_copy(y_hbm.at[pl.ds(0, _CW)],
                                  u0.at[pl.ds(0, _CW)], sem).wait()

        plsc.subcore_barrier()

        @pl.when(sid == 0)
        def _():
            pltpu.sync_copy(s0, out_hbm.at[cid, 0])
            pltpu.sync_copy(s1, out_hbm.at[cid, 1])
            pltpu.sync_copy(s2, out_hbm.at[cid, 2])
            pltpu.sync_copy(s3, out_hbm.at[cid, 3])

    kern = pl.kernel(
        body,
        out_type=jax.ShapeDtypeStruct((_NC, 4, n), jnp.float32),
        mesh=_mesh(),
        scratch_types=[
            pltpu.VMEM((n * 4,), jnp.float32),
            pltpu.VMEM((epw,), jnp.int32),
            pltpu.VMEM((epw,), jnp.int32),
            pltpu.VMEM((epw,), jnp.float32),
            pltpu.VMEM((epw,), jnp.float32),
            pltpu.VMEM((epw,), jnp.float32),
            pltpu.VMEM((epw,), jnp.float32),
            pltpu.VMEM((epw,), jnp.float32),
            pltpu.VMEM_SHARED((n,), jnp.float32),
            pltpu.VMEM_SHARED((n,), jnp.float32),
            pltpu.VMEM_SHARED((n,), jnp.float32),
            pltpu.VMEM_SHARED((n,), jnp.float32),
            pltpu.SemaphoreType.DMA,
        ],
        compiler_params=_sc_params(),
    )
    return kern(y_flat, row, col, w, zeros_n)


def _tail_body(sp_ref, dis_ref, xwT_ref, w4_ref, vo_ref, gcn_b_ref, out_ref):
    dis = dis_ref[...]
    acc = None
    for j in range(xwT_ref.shape[0]):
        hj = jnp.maximum(
            dis * (sp_ref[0, j, :] + sp_ref[1, j, :])
            + dis * dis * xwT_ref[j, :] + gcn_b_ref[j], 0.0)
        sj = jnp.dot(hj[None, :], w4_ref[j],
                     preferred_element_type=jnp.float32)
        acc = sj if acc is None else acc + sj
    out_ref[...] = vo_ref[...] + acc


def _tail(sp, dis, xwT, w4, vo, gcn_b):
    b, ncls = vo.shape
    return pl.pallas_call(
        _tail_body,
        in_specs=[
            pl.BlockSpec(memory_space=pltpu.VMEM),
            pl.BlockSpec(memory_space=pltpu.VMEM),
            pl.BlockSpec(memory_space=pltpu.VMEM),
            pl.BlockSpec(memory_space=pltpu.VMEM),
            pl.BlockSpec(memory_space=pltpu.VMEM),
            pl.BlockSpec(memory_space=pltpu.SMEM),
        ],
        out_shape=jax.ShapeDtypeStruct((b, ncls), jnp.float32),
    )(sp, dis, xwT, w4, vo, gcn_b)


def kernel(num_x, cat_x, edge_index, edge_weights, vanilla_out,
           emb_W, emb_b, gcn_W, gcn_b, fc_W, fc_b):
    n = num_x.shape[0] + cat_x.shape[0]
    e = edge_index.shape[1]
    h = gcn_W.shape[1]
    ncls = fc_b.shape[0]
    k0 = vanilla_out.shape[1]

    span = _NW * _CW
    nch = -(-e // span)
    ep = nch * span
    pad = ep - e
    row = jnp.concatenate([edge_index[0], jnp.zeros((pad,), edge_index.dtype)])
    col = jnp.concatenate([edge_index[1], jnp.zeros((pad,), edge_index.dtype)])
    w = jnp.concatenate([edge_weights, jnp.zeros((pad,), edge_weights.dtype)])
    zeros_n = jnp.zeros((n,), jnp.float32)

    xw_num, xw_cat, vo = _front(num_x, cat_x, emb_W, emb_b, gcn_W,
                                vanilla_out, fc_W[:k0], fc_b)
    xw = jnp.concatenate([xw_num, xw_cat], axis=0)

    degp = _deg(col, w, zeros_n)
    dis, y = _mid(degp, xw)

    sp = _msg(y.reshape(-1), row, col, w, zeros_n)

    w4 = fc_W[k0:].reshape(n, h, ncls).transpose(1, 0, 2)
    return _tail(sp, dis, xw.T, w4, vo, gcn_b)


# speedup vs baseline: 1.3305x; 1.3305x over previous
import dataclasses

import jax
import jax.numpy as jnp
from jax import lax
from jax.experimental import pallas as pl
from jax.experimental.pallas import tpu as pltpu
from jax.experimental.pallas import tpu_sc as plsc

_NC = 2
_NS = 16
_NW = _NC * _NS
_CW = 128
_LANES = 16


def _mesh():
    return plsc.VectorSubcoreMesh(core_axis_name="c", subcore_axis_name="s")


def _sc_params():
    cp = pltpu.CompilerParams()
    if "needs_layout_passes" in pltpu.CompilerParams.__dataclass_fields__:
        cp = dataclasses.replace(cp, needs_layout_passes=False)
    return cp


def _deg(col_e, w_e, zeros_n):
    n = zeros_n.shape[0]
    e = w_e.shape[0]
    epw = e // _NW
    nfull = epw // _CW
    rem = epw - nfull * _CW

    def body(col_hbm, w_hbm, zeros_hbm, out_hbm, col_v, w_v, deg_sh, sem):
        cid = lax.axis_index("c")
        sid = lax.axis_index("s")
        wid = sid * _NC + cid

        @pl.when(sid == 0)
        def _():
            pltpu.sync_copy(zeros_hbm, deg_sh)

        pltpu.sync_copy(col_hbm.at[pl.ds(wid * epw, epw)], col_v)
        pltpu.sync_copy(w_hbm.at[pl.ds(wid * epw, epw)], w_v)
        plsc.subcore_barrier()

        @pl.loop(0, nfull)
        def _(k):
            sl = pl.ds(k * _CW, _CW)
            pltpu.async_copy(w_v.at[sl], deg_sh.at[col_v.at[sl]], sem,
                             add=True)

        if rem:
            sl = pl.ds(nfull * _CW, rem)
            pltpu.async_copy(w_v.at[sl], deg_sh.at[col_v.at[sl]], sem,
                             add=True)

        @pl.loop(0, nfull)
        def _(k):
            pltpu.make_async_copy(w_hbm.at[pl.ds(0, _CW)],
                                  w_v.at[pl.ds(0, _CW)], sem).wait()

        if rem:
            pltpu.make_async_copy(w_hbm.at[pl.ds(0, rem)],
                                  w_v.at[pl.ds(0, rem)], sem).wait()

        plsc.subcore_barrier()

        @pl.when(sid == 0)
        def _():
            pltpu.sync_copy(deg_sh, out_hbm.at[cid])

    kern = pl.kernel(
        body,
        out_type=jax.ShapeDtypeStruct((_NC, n), jnp.float32),
        mesh=_mesh(),
        scratch_types=[
            pltpu.VMEM((epw,), jnp.int32),
            pltpu.VMEM((epw,), jnp.float32),
            pltpu.VMEM_SHARED((n,), jnp.float32),
            pltpu.SemaphoreType.DMA,
        ],
    )
    return kern(col_e, w_e, zeros_n)


def _dense_body(num_x_ref, cat_x_ref, emb_W_ref, emb_b_ref, gcn_W_ref,
                vo_ref, fcW0_ref, fc_b_ref, degp_ref,
                y0_ref, y1_ref, y2_ref, y3_ref, dis_ref, vout_ref):
    cat_emb = (jnp.sum(cat_x_ref[...][:, :, None] * emb_W_ref[...], axis=1)
               + emb_b_ref[...])
    gcn_W = gcn_W_ref[...]
    xw_num = jnp.dot(num_x_ref[...], gcn_W,
                     preferred_element_type=jnp.float32)
    xw_cat = jnp.dot(cat_emb, gcn_W, preferred_element_type=jnp.float32)
    xwT = jnp.transpose(jnp.concatenate([xw_num, xw_cat], axis=0))
    deg = 1.0 + degp_ref[0, :] + degp_ref[1, :]
    dis = lax.rsqrt(deg)
    dis_ref[...] = dis
    y0_ref[...] = dis * xwT[0, :]
    y1_ref[...] = dis * xwT[1, :]
    y2_ref[...] = dis * xwT[2, :]
    y3_ref[...] = dis * xwT[3, :]
    vout_ref[...] = (jnp.dot(vo_ref[...], fcW0_ref[...],
                             preferred_element_type=jnp.float32)
                     + fc_b_ref[...][None, :])


def _dense(num_x, cat_x, emb_W, emb_b, gcn_W, vanilla_out, fcW0, fc_b, degp):
    n = degp.shape[1]
    b, ncls = vanilla_out.shape[0], fcW0.shape[1]
    yv = jax.ShapeDtypeStruct((n,), jnp.float32)
    return pl.pallas_call(
        _dense_body,
        out_shape=(
            yv, yv, yv, yv, yv,
            jax.ShapeDtypeStruct((b, ncls), jnp.float32),
        ),
    )(num_x, cat_x, emb_W, emb_b, gcn_W, vanilla_out, fcW0, fc_b, degp)


def _msg(y0, y1, y2, y3, row_e, col_e, w_e, zeros_n):
    n = y0.shape[0]
    hh = 4
    e = w_e.shape[0]
    epw = e // _NW
    nfull = epw // _CW
    rem = epw - nfull * _CW

    def body(y0_hbm, y1_hbm, y2_hbm, y3_hbm, row_hbm, col_hbm, w_hbm,
             zeros_hbm, out_hbm,
             y0_v, y1_v, y2_v, y3_v, row_v, col_v, w_v,
             u0, u1, u2, u3, s0, s1, s2, s3, sem):
        cid = lax.axis_index("c")
        sid = lax.axis_index("s")
        wid = sid * _NC + cid

        stage = [
            pltpu.async_copy(y0_hbm, y0_v, sem),
            pltpu.async_copy(y1_hbm, y1_v, sem),
            pltpu.async_copy(y2_hbm, y2_v, sem),
            pltpu.async_copy(y3_hbm, y3_v, sem),
            pltpu.async_copy(row_hbm.at[pl.ds(wid * epw, epw)], row_v, sem),
            pltpu.async_copy(col_hbm.at[pl.ds(wid * epw, epw)], col_v, sem),
            pltpu.async_copy(w_hbm.at[pl.ds(wid * epw, epw)], w_v, sem),
        ]

        @pl.when(sid == 0)
        def _():
            pltpu.sync_copy(zeros_hbm, s0)
            pltpu.sync_copy(zeros_hbm, s1)

        @pl.when(sid == 1)
        def _():
            pltpu.sync_copy(zeros_hbm, s2)
            pltpu.sync_copy(zeros_hbm, s3)

        for d in stage:
            d.wait()
        plsc.subcore_barrier()

        def compute(e):
            r16 = row_v[pl.ds(e, _LANES)]
            w16 = w_v[pl.ds(e, _LANES)]
            u0[pl.ds(e, _LANES)] = plsc.load_gather(y0_v, [r16]) * w16
            u1[pl.ds(e, _LANES)] = plsc.load_gather(y1_v, [r16]) * w16
            u2[pl.ds(e, _LANES)] = plsc.load_gather(y2_v, [r16]) * w16
            u3[pl.ds(e, _LANES)] = plsc.load_gather(y3_v, [r16]) * w16

        def scatter(base, width):
            sl = pl.ds(base, width)
            idx = col_v.at[sl]
            pltpu.async_copy(u0.at[sl], s0.at[idx], sem, add=True)
            pltpu.async_copy(u1.at[sl], s1.at[idx], sem, add=True)
            pltpu.async_copy(u2.at[sl], s2.at[idx], sem, add=True)
            pltpu.async_copy(u3.at[sl], s3.at[idx], sem, add=True)

        @pl.loop(0, nfull)
        def _(k):
            @pl.loop(0, _CW, step=_LANES)
            def _(t):
                compute(k * _CW + t)

            scatter(k * _CW, _CW)

        if rem:
            @pl.loop(0, rem, step=_LANES)
            def _(t):
                compute(nfull * _CW + t)

            scatter(nfull * _CW, rem)

        @pl.loop(0, 4 * nfull)
        def _(k):
            pltpu.make_async_copy(w_hbm.at[pl.ds(0, _CW)],
                                  u0.at[pl.ds(0, _CW)], sem).wait()

        if rem:
            for _ in range(4):
                pltpu.make_async_copy(w_hbm.at[pl.ds(0, rem)],
                                      u0.at[pl.ds(0, rem)], sem).wait()

        plsc.subcore_barrier()

        @pl.when(sid == 0)
        def _():
            pltpu.sync_copy(s0, out_hbm.at[cid, 0])
            pltpu.sync_copy(s1, out_hbm.at[cid, 1])

        @pl.when(sid == 1)
        def _():
            pltpu.sync_copy(s2, out_hbm.at[cid, 2])
            pltpu.sync_copy(s3, out_hbm.at[cid, 3])

    kern = pl.kernel(
        body,
        out_type=jax.ShapeDtypeStruct((_NC, hh, n), jnp.float32),
        mesh=_mesh(),
        scratch_types=[
            pltpu.VMEM((n,), jnp.float32),
            pltpu.VMEM((n,), jnp.float32),
            pltpu.VMEM((n,), jnp.float32),
            pltpu.VMEM((n,), jnp.float32),
            pltpu.VMEM((epw,), jnp.int32),
            pltpu.VMEM((epw,), jnp.int32),
            pltpu.VMEM((epw,), jnp.float32),
            pltpu.VMEM((epw,), jnp.float32),
            pltpu.VMEM((epw,), jnp.float32),
            pltpu.VMEM((epw,), jnp.float32),
            pltpu.VMEM((epw,), jnp.float32),
            pltpu.VMEM_SHARED((n,), jnp.float32),
            pltpu.VMEM_SHARED((n,), jnp.float32),
            pltpu.VMEM_SHARED((n,), jnp.float32),
            pltpu.VMEM_SHARED((n,), jnp.float32),
            pltpu.SemaphoreType.DMA,
        ],
        compiler_params=_sc_params(),
    )
    return kern(y0, y1, y2, y3, row_e, col_e, w_e, zeros_n)


def _tail_body(sp_ref, dis_ref, y0_ref, y1_ref, y2_ref, y3_ref,
               w4_ref, vo_ref, gcn_b_ref, out_ref):
    dis = dis_ref[...]
    ys = (y0_ref, y1_ref, y2_ref, y3_ref)
    hs = []
    for j in range(4):
        hs.append(jnp.maximum(
            dis * (sp_ref[0, j, :] + sp_ref[1, j, :] + ys[j][...])
            + gcn_b_ref[j], 0.0))
    hcat = jnp.concatenate(hs, axis=0)
    out_ref[...] = vo_ref[...] + jnp.dot(hcat[None, :], w4_ref[...],
                                         preferred_element_type=jnp.float32)


def _tail(sp, dis, y0, y1, y2, y3, w4, vo, gcn_b):
    b, ncls = vo.shape
    vmem = pl.BlockSpec(memory_space=pltpu.VMEM)
    return pl.pallas_call(
        _tail_body,
        in_specs=[vmem] * 8 + [pl.BlockSpec(memory_space=pltpu.SMEM)],
        out_shape=jax.ShapeDtypeStruct((b, ncls), jnp.float32),
    )(sp, dis, y0, y1, y2, y3, w4, vo, gcn_b)


def kernel(num_x, cat_x, edge_index, edge_weights, vanilla_out,
           emb_W, emb_b, gcn_W, gcn_b, fc_W, fc_b):
    n = num_x.shape[0] + cat_x.shape[0]
    h = gcn_W.shape[1]
    ncls = fc_b.shape[0]
    k0 = vanilla_out.shape[1]

    zeros_n = jnp.zeros((n,), jnp.float32)

    row = edge_index[0]
    col = edge_index[1]
    degp = _deg(col, edge_weights, zeros_n)
    y0, y1, y2, y3, dis, vo = _dense(num_x, cat_x, emb_W, emb_b, gcn_W,
                                     vanilla_out, fc_W[:k0], fc_b, degp)

    sp = _msg(y0, y1, y2, y3, row, col, edge_weights, zeros_n)

    w4 = fc_W[k0:].reshape(n, h, ncls).transpose(1, 0, 2).reshape(n * h, ncls)
    return _tail(sp, dis, y0, y1, y2, y3, w4, vo, gcn_b)


# speedup vs baseline: 1.3827x; 1.0393x over previous
import dataclasses

import jax
import jax.numpy as jnp
from jax import lax
from jax.experimental import pallas as pl
from jax.experimental.pallas import tpu as pltpu
from jax.experimental.pallas import tpu_sc as plsc

_NC = 2
_NS = 16
_NW = _NC * _NS
_CW = 128
_LANES = 16


def _mesh():
    return plsc.VectorSubcoreMesh(core_axis_name="c", subcore_axis_name="s")


def _sc_params():
    cp = pltpu.CompilerParams()
    if "needs_layout_passes" in pltpu.CompilerParams.__dataclass_fields__:
        cp = dataclasses.replace(cp, needs_layout_passes=False)
    return cp


def _deg(col_e, w_e, zeros_n):
    n = zeros_n.shape[0]
    e = w_e.shape[0]
    epw = e // _NW
    nfull = epw // _CW
    rem = epw - nfull * _CW

    def body(col_hbm, w_hbm, zeros_hbm, out_hbm, col_v, w_v, deg_sh, sem):
        cid = lax.axis_index("c")
        sid = lax.axis_index("s")
        wid = sid * _NC + cid

        @pl.when(sid == 0)
        def _():
            pltpu.sync_copy(zeros_hbm, deg_sh)

        pltpu.sync_copy(col_hbm.at[pl.ds(wid * epw, epw)], col_v)
        pltpu.sync_copy(w_hbm.at[pl.ds(wid * epw, epw)], w_v)
        plsc.subcore_barrier()

        @pl.loop(0, nfull)
        def _(k):
            sl = pl.ds(k * _CW, _CW)
            pltpu.async_copy(w_v.at[sl], deg_sh.at[col_v.at[sl]], sem,
                             add=True)

        if rem:
            sl = pl.ds(nfull * _CW, rem)
            pltpu.async_copy(w_v.at[sl], deg_sh.at[col_v.at[sl]], sem,
                             add=True)

        @pl.loop(0, nfull)
        def _(k):
            pltpu.make_async_copy(w_hbm.at[pl.ds(0, _CW)],
                                  w_v.at[pl.ds(0, _CW)], sem).wait()

        if rem:
            pltpu.make_async_copy(w_hbm.at[pl.ds(0, rem)],
                                  w_v.at[pl.ds(0, rem)], sem).wait()

        plsc.subcore_barrier()

        @pl.when(sid == 0)
        def _():
            pltpu.sync_copy(deg_sh, out_hbm.at[cid])

    kern = pl.kernel(
        body,
        out_type=jax.ShapeDtypeStruct((_NC, n), jnp.float32),
        mesh=_mesh(),
        scratch_types=[
            pltpu.VMEM((epw,), jnp.int32),
            pltpu.VMEM((epw,), jnp.float32),
            pltpu.VMEM_SHARED((n,), jnp.float32),
            pltpu.SemaphoreType.DMA,
        ],
    )
    return kern(col_e, w_e, zeros_n)


def _dense_body(num_x_ref, cat_x_ref, emb_W_ref, emb_b_ref, gcn_W_ref,
                vo_ref, fcW0_ref, fc_b_ref, degp_ref,
                y0_ref, y1_ref, y2_ref, y3_ref, dis_ref, vout_ref):
    cat_emb = (jnp.sum(cat_x_ref[...][:, :, None] * emb_W_ref[...], axis=1)
               + emb_b_ref[...])
    gcn_W = gcn_W_ref[...]
    xw_num = jnp.dot(num_x_ref[...], gcn_W,
                     preferred_element_type=jnp.float32)
    xw_cat = jnp.dot(cat_emb, gcn_W, preferred_element_type=jnp.float32)
    xwT = jnp.transpose(jnp.concatenate([xw_num, xw_cat], axis=0))
    deg = 1.0 + degp_ref[0, :] + degp_ref[1, :]
    dis = lax.rsqrt(deg)
    dis_ref[...] = dis
    y0_ref[...] = dis * xwT[0, :]
    y1_ref[...] = dis * xwT[1, :]
    y2_ref[...] = dis * xwT[2, :]
    y3_ref[...] = dis * xwT[3, :]
    vout_ref[...] = (jnp.dot(vo_ref[...], fcW0_ref[...],
                             preferred_element_type=jnp.float32)
                     + fc_b_ref[...][None, :])


def _dense(num_x, cat_x, emb_W, emb_b, gcn_W, vanilla_out, fcW0, fc_b, degp):
    n = degp.shape[1]
    b, ncls = vanilla_out.shape[0], fcW0.shape[1]
    yv = jax.ShapeDtypeStruct((n,), jnp.float32)
    return pl.pallas_call(
        _dense_body,
        out_shape=(
            yv, yv, yv, yv, yv,
            jax.ShapeDtypeStruct((b, ncls), jnp.float32),
        ),
    )(num_x, cat_x, emb_W, emb_b, gcn_W, vanilla_out, fcW0, fc_b, degp)


def _msg(y0, y1, y2, y3, row_e, col_e, w_e, zeros_n):
    n = y0.shape[0]
    hh = 4
    e = w_e.shape[0]
    epw = e // _NW
    nfull = epw // _CW
    rem = epw - nfull * _CW

    def body(y0_hbm, y1_hbm, y2_hbm, y3_hbm, row_hbm, col_hbm, w_hbm,
             zeros_hbm, out_hbm,
             y0_v, y1_v, y2_v, y3_v, row_v, col_v, w_v,
             u0, u1, u2, u3, s0, s1, s2, s3, sem):
        cid = lax.axis_index("c")
        sid = lax.axis_index("s")
        wid = sid * _NC + cid

        stage = [
            pltpu.async_copy(y0_hbm, y0_v, sem),
            pltpu.async_copy(y1_hbm, y1_v, sem),
            pltpu.async_copy(y2_hbm, y2_v, sem),
            pltpu.async_copy(y3_hbm, y3_v, sem),
            pltpu.async_copy(row_hbm.at[pl.ds(wid * epw, epw)], row_v, sem),
            pltpu.async_copy(col_hbm.at[pl.ds(wid * epw, epw)], col_v, sem),
            pltpu.async_copy(w_hbm.at[pl.ds(wid * epw, epw)], w_v, sem),
        ]

        @pl.when(sid == 0)
        def _():
            pltpu.sync_copy(zeros_hbm, s0)
            pltpu.sync_copy(zeros_hbm, s1)

        @pl.when(sid == 1)
        def _():
            pltpu.sync_copy(zeros_hbm, s2)
            pltpu.sync_copy(zeros_hbm, s3)

        for d in stage:
            d.wait()
        plsc.subcore_barrier()

        def compute(e):
            r16 = row_v[pl.ds(e, _LANES)]
            w16 = w_v[pl.ds(e, _LANES)]
            u0[pl.ds(e, _LANES)] = plsc.load_gather(y0_v, [r16]) * w16
            u1[pl.ds(e, _LANES)] = plsc.load_gather(y1_v, [r16]) * w16
            u2[pl.ds(e, _LANES)] = plsc.load_gather(y2_v, [r16]) * w16
            u3[pl.ds(e, _LANES)] = plsc.load_gather(y3_v, [r16]) * w16

        def scatter(base, width):
            sl = pl.ds(base, width)
            idx = col_v.at[sl]
            pltpu.async_copy(u0.at[sl], s0.at[idx], sem, add=True)
            pltpu.async_copy(u1.at[sl], s1.at[idx], sem, add=True)
            pltpu.async_copy(u2.at[sl], s2.at[idx], sem, add=True)
            pltpu.async_copy(u3.at[sl], s3.at[idx], sem, add=True)

        @pl.loop(0, nfull)
        def _(k):
            @pl.loop(0, _CW, step=_LANES)
            def _(t):
                compute(k * _CW + t)

            scatter(k * _CW, _CW)

        if rem:
            @pl.loop(0, rem, step=_LANES)
            def _(t):
                compute(nfull * _CW + t)

            scatter(nfull * _CW, rem)

        @pl.loop(0, 4 * nfull)
        def _(k):
            pltpu.make_async_copy(w_hbm.at[pl.ds(0, _CW)],
                                  u0.at[pl.ds(0, _CW)], sem).wait()

        if rem:
            for _ in range(4):
                pltpu.make_async_copy(w_hbm.at[pl.ds(0, rem)],
                                      u0.at[pl.ds(0, rem)], sem).wait()

        plsc.subcore_barrier()

        @pl.when(sid == 0)
        def _():
            pltpu.sync_copy(s0, out_hbm.at[cid, 0])
            pltpu.sync_copy(s1, out_hbm.at[cid, 1])

        @pl.when(sid == 1)
        def _():
            pltpu.sync_copy(s2, out_hbm.at[cid, 2])
            pltpu.sync_copy(s3, out_hbm.at[cid, 3])

    kern = pl.kernel(
        body,
        out_type=jax.ShapeDtypeStruct((_NC, hh, n), jnp.float32),
        mesh=_mesh(),
        scratch_types=[
            pltpu.VMEM((n,), jnp.float32),
            pltpu.VMEM((n,), jnp.float32),
            pltpu.VMEM((n,), jnp.float32),
            pltpu.VMEM((n,), jnp.float32),
            pltpu.VMEM((epw,), jnp.int32),
            pltpu.VMEM((epw,), jnp.int32),
            pltpu.VMEM((epw,), jnp.float32),
            pltpu.VMEM((epw,), jnp.float32),
            pltpu.VMEM((epw,), jnp.float32),
            pltpu.VMEM((epw,), jnp.float32),
            pltpu.VMEM((epw,), jnp.float32),
            pltpu.VMEM_SHARED((n,), jnp.float32),
            pltpu.VMEM_SHARED((n,), jnp.float32),
            pltpu.VMEM_SHARED((n,), jnp.float32),
            pltpu.VMEM_SHARED((n,), jnp.float32),
            pltpu.SemaphoreType.DMA,
        ],
        compiler_params=_sc_params(),
    )
    return kern(y0, y1, y2, y3, row_e, col_e, w_e, zeros_n)


def _tail_body(sp_ref, dis_ref, y0_ref, y1_ref, y2_ref, y3_ref,
               w40_ref, vo_ref, gcn_b_ref, out_ref):
    dis = dis_ref[...]
    ys = (y0_ref, y1_ref, y2_ref, y3_ref)
    ncls = out_ref.shape[1]
    acc = None
    for j in range(4):
        hj = jnp.maximum(
            dis * (sp_ref[0, j, :] + sp_ref[1, j, :] + ys[j][...])
            + gcn_b_ref[j], 0.0)
        wj = w40_ref[:, j * ncls:(j + 1) * ncls]
        sj = jnp.dot(hj[None, :], wj, preferred_element_type=jnp.float32)
        acc = sj if acc is None else acc + sj
    out_ref[...] = vo_ref[...] + acc


def _tail(sp, dis, y0, y1, y2, y3, w40, vo, gcn_b):
    b, ncls = vo.shape
    vmem = pl.BlockSpec(memory_space=pltpu.VMEM)
    return pl.pallas_call(
        _tail_body,
        in_specs=[vmem] * 8 + [pl.BlockSpec(memory_space=pltpu.SMEM)],
        out_shape=jax.ShapeDtypeStruct((b, ncls), jnp.float32),
    )(sp, dis, y0, y1, y2, y3, w40, vo, gcn_b)


def kernel(num_x, cat_x, edge_index, edge_weights, vanilla_out,
           emb_W, emb_b, gcn_W, gcn_b, fc_W, fc_b):
    n = num_x.shape[0] + cat_x.shape[0]
    h = gcn_W.shape[1]
    ncls = fc_b.shape[0]
    k0 = vanilla_out.shape[1]

    zeros_n = jnp.zeros((n,), jnp.float32)

    row = edge_index[0]
    col = edge_index[1]
    degp = _deg(col, edge_weights, zeros_n)
    y0, y1, y2, y3, dis, vo = _dense(num_x, cat_x, emb_W, emb_b, gcn_W,
                                     vanilla_out, fc_W[:k0], fc_b, degp)

    sp = _msg(y0, y1, y2, y3, row, col, edge_weights, zeros_n)

    w40 = fc_W[k0:].reshape(n, h * ncls) + 0.0 * degp[0:1, 0:1]
    return _tail(sp, dis, y0, y1, y2, y3, w40, vo, gcn_b)
